# baseline (speedup 1.0000x reference)
import jax
import jax.numpy as jnp
from jax import lax
from jax.experimental import pallas as pl
from jax.experimental.pallas import tpu as pltpu
from jax.experimental.pallas import tpu_sc as plsc

_VOCAB = 1000000
_D = 64
_H = 128
_B = 4096
_L = 200

_NC = 2
_NS = 16
_NW = _NC * _NS
_RPT = _B // _NW
_F = 2 * _L
_HW = _D // 2
_CHUNKS = ((0, 128), (128, 128), (256, 128), (384, 16))


def _sc_body(idx_hbm, tab_hbm, out_hbm, idx_v, pa, pb, ra, rb, out_v,
             s_a, s_b):
    c = lax.axis_index("c")
    s = lax.axis_index("s")
    wid = s * _NC + c
    rbase = wid * _RPT

    pltpu.sync_copy(idx_hbm.at[pl.ds(rbase, _RPT)], idx_v)

    lane = lax.broadcasted_iota(jnp.int32, (16,), 0)

    def precompute(row, pv):
        def pg(g, carry):
            iv = idx_v[row, pl.ds(g * 16, 16)]
            dbl = iv + iv
            pos = (g * 32) + lane + lane
            plsc.store_scatter(pv, [pos], dbl)
            plsc.store_scatter(pv, [pos + 1], dbl + 1)
            return carry
        lax.fori_loop(0, _L // 16, pg, 0)
        iv = idx_v[row, pl.ds(_L - 16, 16)]
        dbl = iv + iv
        pos = (2 * (_L - 16)) + lane + lane
        plsc.store_scatter(pv, [pos], dbl)
        plsc.store_scatter(pv, [pos + 1], dbl + 1)
        return None

    def gathers(pv, buf, sem):
        for off, width in _CHUNKS:
            pltpu.async_copy(
                tab_hbm.at[pv.at[pl.ds(off, width)]],
                buf.at[pl.ds(off, width)], sem)

    def waits(pv, buf, sem):
        for off, width in _CHUNKS:
            pltpu.make_async_copy(
                tab_hbm.at[pv.at[pl.ds(off, width)]],
                buf.at[pl.ds(off, width)], sem).wait()

    zero = jnp.zeros((16,), jnp.float32)
    acc_init = (zero,) * 8

    def reduce_row(buf, accs):
        def rb(i, a):
            r0 = 4 * i
            return (
                a[0] + buf[r0, pl.ds(0, 16)],
                a[1] + buf[r0, pl.ds(16, 16)],
                a[2] + buf[r0 + 1, pl.ds(0, 16)],
                a[3] + buf[r0 + 1, pl.ds(16, 16)],
                a[4] + buf[r0 + 2, pl.ds(0, 16)],
                a[5] + buf[r0 + 2, pl.ds(16, 16)],
                a[6] + buf[r0 + 3, pl.ds(0, 16)],
                a[7] + buf[r0 + 3, pl.ds(16, 16)],
            )
        return lax.fori_loop(0, _L // 2, rb, accs)

    def store_row(row, accs):
        out_v[row, pl.ds(0, 16)] = accs[0] + accs[4]
        out_v[row, pl.ds(16, 16)] = accs[1] + accs[5]
        out_v[row, pl.ds(32, 16)] = accs[2] + accs[6]
        out_v[row, pl.ds(48, 16)] = accs[3] + accs[7]

    precompute(0, pa)
    gathers(pa, ra, s_a)
    precompute(1, pb)
    gathers(pb, rb, s_b)

    nbb = _RPT // 2

    def iter_body(bb, carry):
        r0 = 2 * bb
        r1 = r0 + 1
        more = bb < nbb - 1

        waits(pa, ra, s_a)
        acc = reduce_row(ra, acc_init)

        @pl.when(more)
        def _():
            precompute(r0 + 2, pa)
            gathers(pa, ra, s_a)

        store_row(r0, acc)

        waits(pb, rb, s_b)
        acc = reduce_row(rb, acc_init)

        @pl.when(more)
        def _():
            precompute(r1 + 2, pb)
            gathers(pb, rb, s_b)

        store_row(r1, acc)
        return carry

    lax.fori_loop(0, nbb, iter_body, 0)

    pltpu.sync_copy(out_v, out_hbm.at[pl.ds(rbase, _RPT)])


_sc_gather_sum = pl.kernel(
    _sc_body,
    out_type=jax.ShapeDtypeStruct((_B, _D), jnp.float32),
    mesh=plsc.VectorSubcoreMesh(core_axis_name="c", subcore_axis_name="s",
                                num_cores=_NC, num_subcores=_NS),
    scratch_types=[
        pltpu.VMEM((_RPT, _L), jnp.int32),
        pltpu.VMEM((_F,), jnp.int32),
        pltpu.VMEM((_F,), jnp.int32),
        pltpu.VMEM((_F, _HW), jnp.float32),
        pltpu.VMEM((_F, _HW), jnp.float32),
        pltpu.VMEM((_RPT, _D), jnp.float32),
        pltpu.SemaphoreType.DMA,
        pltpu.SemaphoreType.DMA,
    ],
    compiler_params=pltpu.CompilerParams(use_tc_tiling_on_sc=False,
                                         needs_layout_passes=False),
)


_NBLK = 7813


def _tr_body(tabt_hbm, out_hbm, ba, bb, oa, ob, s_a, s_b, s_oa, s_ob):
    c = lax.axis_index("c")
    s = lax.axis_index("s")
    wid = s * _NC + c

    lane = lax.broadcasted_iota(jnp.int32, (16,), 0)

    def fetch(blk, buf, sem):
        pltpu.async_copy(tabt_hbm.at[:, pl.ds(blk * 128, 128)], buf, sem)

    def fwait(blk, buf, sem):
        pltpu.make_async_copy(
            tabt_hbm.at[:, pl.ds(0, 128)], buf, sem).wait()

    def transpose(buf, obuf):
        dims = [lane + 16 * m for m in range(4)]
        for sft in range(16):
            rot = (lane + sft) & 15
            st_r = lax.shift_right_logical(rot, 1)
            st_c = (rot & 1) * 64 + lane
            stc = [st_c + 16 * m for m in range(4)]

            def inner(e0g, carry, rot=rot, st_r=st_r, stc=stc):
                ld_c = rot + e0g * 16
                st_rr = st_r + e0g * 8
                for m in range(4):
                    v = plsc.load_gather(buf, [dims[m], ld_c])
                    plsc.store_scatter(obuf, [st_rr, stc[m]], v)
                return carry
            lax.fori_loop(0, 8, inner, 0)

    def owrite(blk, obuf, sem):
        pltpu.async_copy(obuf, out_hbm.at[pl.ds(blk * 64, 64)], sem)

    def owait(obuf, sem):
        pltpu.make_async_copy(
            obuf, out_hbm.at[pl.ds(0, 64)], sem).wait()

    nit = 123

    fetch(wid, ba, s_a)

    def it_body(k, carry):
        c0 = wid + 64 * k
        c1 = c0 + 32

        @pl.when(c0 < 7812)
        def _():
            fwait(c0, ba, s_a)

            @pl.when(c1 < 7812)
            def _():
                fetch(c1, bb, s_b)

            @pl.when(k > 0)
            def _():
                owait(oa, s_oa)
            transpose(ba, oa)
            owrite(c0, oa, s_oa)

        @pl.when(c1 < 7812)
        def _():
            fwait(c1, bb, s_b)
            nxt = c0 + 64

            @pl.when(nxt < 7812)
            def _():
                fetch(nxt, ba, s_a)

            @pl.when(k > 0)
            def _():
                owait(ob, s_ob)
            transpose(bb, ob)
            owrite(c1, ob, s_ob)
        return carry

    lax.fori_loop(0, nit, it_body, 0)

    owait(oa, s_oa)
    owait(ob, s_ob)


_sc_densify = pl.kernel(
    _tr_body,
    out_type=jax.ShapeDtypeStruct((_VOCAB // 2, 2 * _D), jnp.float32),
    mesh=plsc.VectorSubcoreMesh(core_axis_name="c", subcore_axis_name="s",
                                num_cores=_NC, num_subcores=_NS),
    scratch_types=[
        pltpu.VMEM((_D, 128), jnp.float32),
        pltpu.VMEM((_D, 128), jnp.float32),
        pltpu.VMEM((_D, 128), jnp.float32),
        pltpu.VMEM((_D, 128), jnp.float32),
        pltpu.SemaphoreType.DMA,
        pltpu.SemaphoreType.DMA,
        pltpu.SemaphoreType.DMA,
        pltpu.SemaphoreType.DMA,
    ],
    compiler_params=pltpu.CompilerParams(needs_layout_passes=False),
)


def _mlp_body(h_ref, w1_ref, b1_ref, w2_ref, b2_ref, o_ref):
    h = h_ref[...] * (1.0 / _L)
    z = jnp.dot(h, w1_ref[...], preferred_element_type=jnp.float32)
    z = jnp.maximum(z + b1_ref[...], 0.0)
    o_ref[...] = jnp.dot(z, w2_ref[...],
                         preferred_element_type=jnp.float32) + b2_ref[...]


_mlp_call = pl.pallas_call(
    _mlp_body,
    out_shape=jax.ShapeDtypeStruct((_B, _H), jnp.float32),
)


def kernel(x, emb_table, W1, b1, W2, b2):
    t2 = _sc_densify(emb_table.T)
    t_tail = emb_table[7812 * 128:].reshape(32, 2 * _D)
    t2 = jax.lax.dynamic_update_slice(t2, t_tail, (7812 * 64, 0))
    t8 = t2.reshape(2 * _VOCAB, _HW)
    sums = _sc_gather_sum(x, t8)
    w2p = jnp.zeros((_H, _H), jnp.float32).at[:, :2].set(W2)
    b2p = jnp.zeros((1, _H), jnp.float32).at[0, :2].set(b2)
    out = _mlp_call(sums, W1, b1.reshape(1, _H), w2p, b2p)
    return out[:, :2]


# speedup vs baseline: 1.3544x; 1.3544x over previous
import jax
import jax.numpy as jnp
from jax import lax
from jax.experimental import pallas as pl
from jax.experimental.pallas import tpu as pltpu
from jax.experimental.pallas import tpu_sc as plsc

_VOCAB = 1000000
_D = 64
_H = 128
_B = 4096
_L = 200

_NC = 2
_NS = 16
_NW = _NC * _NS
_RPT = _B // _NW
_F = 2 * _L
_HW = _D // 2
_CHUNKS = ((0, 128), (128, 128), (256, 128), (384, 16))


def _sc_body(idx_hbm, tab_hbm, out_hbm, idx_v, pa, pb, ra, rb, out_v,
             s_a, s_b):
    c = lax.axis_index("c")
    s = lax.axis_index("s")
    wid = s * _NC + c
    rbase = wid * _RPT

    pltpu.sync_copy(idx_hbm.at[pl.ds(rbase, _RPT)], idx_v)

    lane = lax.broadcasted_iota(jnp.int32, (16,), 0)

    def precompute(row, pv):
        def pg(g, carry):
            iv = idx_v[row, pl.ds(g * 16, 16)]
            dbl = iv + iv
            pos = (g * 32) + lane + lane
            plsc.store_scatter(pv, [pos], dbl)
            plsc.store_scatter(pv, [pos + 1], dbl + 1)
            return carry
        lax.fori_loop(0, _L // 16, pg, 0)
        iv = idx_v[row, pl.ds(_L - 16, 16)]
        dbl = iv + iv
        pos = (2 * (_L - 16)) + lane + lane
        plsc.store_scatter(pv, [pos], dbl)
        plsc.store_scatter(pv, [pos + 1], dbl + 1)
        return None

    def gathers(pv, buf, sem):
        for off, width in _CHUNKS:
            pltpu.async_copy(
                tab_hbm.at[pv.at[pl.ds(off, width)]],
                buf.at[pl.ds(off, width)], sem)

    def waits(pv, buf, sem):
        for off, width in _CHUNKS:
            pltpu.make_async_copy(
                tab_hbm.at[pv.at[pl.ds(off, width)]],
                buf.at[pl.ds(off, width)], sem).wait()

    zero = jnp.zeros((16,), jnp.float32)
    acc_init = (zero,) * 8

    def reduce_row(buf, accs):
        def rb(i, a):
            r0 = 4 * i
            return (
                a[0] + buf[r0, pl.ds(0, 16)],
                a[1] + buf[r0, pl.ds(16, 16)],
                a[2] + buf[r0 + 1, pl.ds(0, 16)],
                a[3] + buf[r0 + 1, pl.ds(16, 16)],
                a[4] + buf[r0 + 2, pl.ds(0, 16)],
                a[5] + buf[r0 + 2, pl.ds(16, 16)],
                a[6] + buf[r0 + 3, pl.ds(0, 16)],
                a[7] + buf[r0 + 3, pl.ds(16, 16)],
            )
        return lax.fori_loop(0, _L // 2, rb, accs)

    def store_row(row, accs):
        out_v[row, pl.ds(0, 16)] = accs[0] + accs[4]
        out_v[row, pl.ds(16, 16)] = accs[1] + accs[5]
        out_v[row, pl.ds(32, 16)] = accs[2] + accs[6]
        out_v[row, pl.ds(48, 16)] = accs[3] + accs[7]

    precompute(0, pa)
    gathers(pa, ra, s_a)
    precompute(1, pb)
    gathers(pb, rb, s_b)

    nbb = _RPT // 2

    def iter_body(bb, carry):
        r0 = 2 * bb
        r1 = r0 + 1
        more = bb < nbb - 1

        waits(pa, ra, s_a)
        acc = reduce_row(ra, acc_init)

        @pl.when(more)
        def _():
            precompute(r0 + 2, pa)
            gathers(pa, ra, s_a)

        store_row(r0, acc)

        waits(pb, rb, s_b)
        acc = reduce_row(rb, acc_init)

        @pl.when(more)
        def _():
            precompute(r1 + 2, pb)
            gathers(pb, rb, s_b)

        store_row(r1, acc)
        return carry

    lax.fori_loop(0, nbb, iter_body, 0)

    pltpu.sync_copy(out_v, out_hbm.at[pl.ds(rbase, _RPT)])


_sc_gather_sum = pl.kernel(
    _sc_body,
    out_type=jax.ShapeDtypeStruct((_B, _D), jnp.float32),
    mesh=plsc.VectorSubcoreMesh(core_axis_name="c", subcore_axis_name="s",
                                num_cores=_NC, num_subcores=_NS),
    scratch_types=[
        pltpu.VMEM((_RPT, _L), jnp.int32),
        pltpu.VMEM((_F,), jnp.int32),
        pltpu.VMEM((_F,), jnp.int32),
        pltpu.VMEM((_F, _HW), jnp.float32),
        pltpu.VMEM((_F, _HW), jnp.float32),
        pltpu.VMEM((_RPT, _D), jnp.float32),
        pltpu.SemaphoreType.DMA,
        pltpu.SemaphoreType.DMA,
    ],
    compiler_params=pltpu.CompilerParams(use_tc_tiling_on_sc=False,
                                         needs_layout_passes=False),
)


_NBLK = 7813


def _tr_body(tabt_hbm, out_hbm, ba, bb, oa, ob, s_a, s_b, s_oa, s_ob):
    c = lax.axis_index("c")
    s = lax.axis_index("s")
    wid = s * _NC + c

    lane = lax.broadcasted_iota(jnp.int32, (16,), 0)

    def fetch(blk, buf, sem):
        pltpu.async_copy(tabt_hbm.at[:, pl.ds(blk * 128, 128)], buf, sem)

    def fwait(blk, buf, sem):
        pltpu.make_async_copy(
            tabt_hbm.at[:, pl.ds(0, 128)], buf, sem).wait()

    def transpose(buf, obuf):
        dims = [lane + 16 * m for m in range(4)]
        for sft in range(16):
            rot = (lane + sft) & 15
            st_r = lax.shift_right_logical(rot, 1)
            st_c = (rot & 1) * 64 + lane
            stc = [st_c + 16 * m for m in range(4)]

            @plsc.parallel_loop(0, 8, unroll=2)
            def _(e0g, rot=rot, st_r=st_r, stc=stc):
                ld_c = rot + e0g * 16
                st_rr = st_r + e0g * 8
                for m in range(4):
                    v = plsc.load_gather(buf, [dims[m], ld_c])
                    plsc.store_scatter(obuf, [st_rr, stc[m]], v)

    def owrite(blk, obuf, sem):
        pltpu.async_copy(obuf, out_hbm.at[pl.ds(blk * 64, 64)], sem)

    def owait(obuf, sem):
        pltpu.make_async_copy(
            obuf, out_hbm.at[pl.ds(0, 64)], sem).wait()

    nit = 123

    fetch(wid, ba, s_a)

    def it_body(k, carry):
        c0 = wid + 64 * k
        c1 = c0 + 32

        @pl.when(c0 < 7812)
        def _():
            fwait(c0, ba, s_a)

            @pl.when(c1 < 7812)
            def _():
                fetch(c1, bb, s_b)

            @pl.when(k > 0)
            def _():
                owait(oa, s_oa)
            transpose(ba, oa)
            owrite(c0, oa, s_oa)

        @pl.when(c1 < 7812)
        def _():
            fwait(c1, bb, s_b)
            nxt = c0 + 64

            @pl.when(nxt < 7812)
            def _():
                fetch(nxt, ba, s_a)

            @pl.when(k > 0)
            def _():
                owait(ob, s_ob)
            transpose(bb, ob)
            owrite(c1, ob, s_ob)
        return carry

    lax.fori_loop(0, nit, it_body, 0)

    owait(oa, s_oa)
    owait(ob, s_ob)


_sc_densify = pl.kernel(
    _tr_body,
    out_type=jax.ShapeDtypeStruct((_VOCAB // 2, 2 * _D), jnp.float32),
    mesh=plsc.VectorSubcoreMesh(core_axis_name="c", subcore_axis_name="s",
                                num_cores=_NC, num_subcores=_NS),
    scratch_types=[
        pltpu.VMEM((_D, 128), jnp.float32),
        pltpu.VMEM((_D, 128), jnp.float32),
        pltpu.VMEM((_D, 128), jnp.float32),
        pltpu.VMEM((_D, 128), jnp.float32),
        pltpu.SemaphoreType.DMA,
        pltpu.SemaphoreType.DMA,
        pltpu.SemaphoreType.DMA,
        pltpu.SemaphoreType.DMA,
    ],
    compiler_params=pltpu.CompilerParams(needs_layout_passes=False),
)


def _mlp_body(h_ref, w1_ref, b1_ref, w2_ref, b2_ref, o_ref):
    h = h_ref[...] * (1.0 / _L)
    z = jnp.dot(h, w1_ref[...], preferred_element_type=jnp.float32)
    z = jnp.maximum(z + b1_ref[...], 0.0)
    o_ref[...] = jnp.dot(z, w2_ref[...],
                         preferred_element_type=jnp.float32) + b2_ref[...]


_mlp_call = pl.pallas_call(
    _mlp_body,
    out_shape=jax.ShapeDtypeStruct((_B, _H), jnp.float32),
)


def kernel(x, emb_table, W1, b1, W2, b2):
    t2 = _sc_densify(emb_table.T)
    t_tail = emb_table[7812 * 128:].reshape(32, 2 * _D)
    t2 = jax.lax.dynamic_update_slice(t2, t_tail, (7812 * 64, 0))
    t8 = t2.reshape(2 * _VOCAB, _HW)
    sums = _sc_gather_sum(x, t8)
    w2p = jnp.zeros((_H, _H), jnp.float32).at[:, :2].set(W2)
    b2p = jnp.zeros((1, _H), jnp.float32).at[0, :2].set(b2)
    out = _mlp_call(sums, W1, b1.reshape(1, _H), w2p, b2p)
    return out[:, :2]


# speedup vs baseline: 1.5656x; 1.1560x over previous
import jax
import jax.numpy as jnp
from jax import lax
from jax.experimental import pallas as pl
from jax.experimental.pallas import tpu as pltpu
from jax.experimental.pallas import tpu_sc as plsc

_VOCAB = 1000000
_D = 64
_H = 128
_B = 4096
_L = 200

_NC = 2
_NS = 16
_NW = _NC * _NS
_RPT = _B // _NW
_F = 2 * _L
_HW = _D // 2
_CHUNKS = ((0, 128), (128, 128), (256, 128), (384, 16))


def _sc_body(idx_hbm, tab_hbm, out_hbm, idx_v, pa, pb, ra, rb, out_v,
             s_a, s_b):
    c = lax.axis_index("c")
    s = lax.axis_index("s")
    wid = s * _NC + c
    rbase = wid * _RPT

    pltpu.sync_copy(idx_hbm.at[pl.ds(rbase, _RPT)], idx_v)

    lane = lax.broadcasted_iota(jnp.int32, (16,), 0)

    def precompute(row, pv):
        def pg(g, carry):
            iv = idx_v[row, pl.ds(g * 16, 16)]
            dbl = iv + iv
            pos = (g * 32) + lane + lane
            plsc.store_scatter(pv, [pos], dbl)
            plsc.store_scatter(pv, [pos + 1], dbl + 1)
            return carry
        lax.fori_loop(0, _L // 16, pg, 0)
        iv = idx_v[row, pl.ds(_L - 16, 16)]
        dbl = iv + iv
        pos = (2 * (_L - 16)) + lane + lane
        plsc.store_scatter(pv, [pos], dbl)
        plsc.store_scatter(pv, [pos + 1], dbl + 1)
        return None

    def gathers(pv, buf, sem):
        for off, width in _CHUNKS:
            pltpu.async_copy(
                tab_hbm.at[pv.at[pl.ds(off, width)]],
                buf.at[pl.ds(off, width)], sem)

    def waits(pv, buf, sem):
        for off, width in _CHUNKS:
            pltpu.make_async_copy(
                tab_hbm.at[pv.at[pl.ds(off, width)]],
                buf.at[pl.ds(off, width)], sem).wait()

    zero = jnp.zeros((16,), jnp.float32)
    acc_init = (zero,) * 8

    def reduce_row(buf, accs):
        def rb(i, a):
            r0 = 4 * i
            return (
                a[0] + buf[r0, pl.ds(0, 16)],
                a[1] + buf[r0, pl.ds(16, 16)],
                a[2] + buf[r0 + 1, pl.ds(0, 16)],
                a[3] + buf[r0 + 1, pl.ds(16, 16)],
                a[4] + buf[r0 + 2, pl.ds(0, 16)],
                a[5] + buf[r0 + 2, pl.ds(16, 16)],
                a[6] + buf[r0 + 3, pl.ds(0, 16)],
                a[7] + buf[r0 + 3, pl.ds(16, 16)],
            )
        return lax.fori_loop(0, _L // 2, rb, accs)

    def store_row(row, accs):
        out_v[row, pl.ds(0, 16)] = accs[0] + accs[4]
        out_v[row, pl.ds(16, 16)] = accs[1] + accs[5]
        out_v[row, pl.ds(32, 16)] = accs[2] + accs[6]
        out_v[row, pl.ds(48, 16)] = accs[3] + accs[7]

    precompute(0, pa)
    gathers(pa, ra, s_a)
    precompute(1, pb)
    gathers(pb, rb, s_b)

    nbb = _RPT // 2

    def iter_body(bb, carry):
        r0 = 2 * bb
        r1 = r0 + 1
        more = bb < nbb - 1

        waits(pa, ra, s_a)
        acc = reduce_row(ra, acc_init)

        @pl.when(more)
        def _():
            precompute(r0 + 2, pa)
            gathers(pa, ra, s_a)

        store_row(r0, acc)

        waits(pb, rb, s_b)
        acc = reduce_row(rb, acc_init)

        @pl.when(more)
        def _():
            precompute(r1 + 2, pb)
            gathers(pb, rb, s_b)

        store_row(r1, acc)
        return carry

    lax.fori_loop(0, nbb, iter_body, 0)

    pltpu.sync_copy(out_v, out_hbm.at[pl.ds(rbase, _RPT)])


_sc_gather_sum = pl.kernel(
    _sc_body,
    out_type=jax.ShapeDtypeStruct((_B, _D), jnp.float32),
    mesh=plsc.VectorSubcoreMesh(core_axis_name="c", subcore_axis_name="s",
                                num_cores=_NC, num_subcores=_NS),
    scratch_types=[
        pltpu.VMEM((_RPT, _L), jnp.int32),
        pltpu.VMEM((_F,), jnp.int32),
        pltpu.VMEM((_F,), jnp.int32),
        pltpu.VMEM((_F, _HW), jnp.float32),
        pltpu.VMEM((_F, _HW), jnp.float32),
        pltpu.VMEM((_RPT, _D), jnp.float32),
        pltpu.SemaphoreType.DMA,
        pltpu.SemaphoreType.DMA,
    ],
    compiler_params=pltpu.CompilerParams(use_tc_tiling_on_sc=False,
                                         needs_layout_passes=False),
)


_NBLK = 7813


def _tr_body(tabt_hbm, out_hbm, ba, bb, oa, ob, s_a, s_b, s_oa, s_ob):
    c = lax.axis_index("c")
    s = lax.axis_index("s")
    wid = s * _NC + c

    lane = lax.broadcasted_iota(jnp.int32, (16,), 0)

    def fetch(blk, buf, sem):
        pltpu.async_copy(tabt_hbm.at[:, pl.ds(blk * 128, 128)], buf, sem)

    def fwait(blk, buf, sem):
        pltpu.make_async_copy(
            tabt_hbm.at[:, pl.ds(0, 128)], buf, sem).wait()

    def transpose(buf, obuf):
        dims = [lane + 16 * m for m in range(4)]
        for sft in range(16):
            rot = (lane + sft) & 15
            st_r = lax.shift_right_logical(rot, 1)
            st_c = (rot & 1) * 64 + lane
            stc = [st_c + 16 * m for m in range(4)]

            @plsc.parallel_loop(0, 8, unroll=4)
            def _(e0g, rot=rot, st_r=st_r, stc=stc):
                ld_c = rot + e0g * 16
                st_rr = st_r + e0g * 8
                for m in range(4):
                    v = plsc.load_gather(buf, [dims[m], ld_c])
                    plsc.store_scatter(obuf, [st_rr, stc[m]], v)

    def owrite(blk, obuf, sem):
        pltpu.async_copy(obuf, out_hbm.at[pl.ds(blk * 64, 64)], sem)

    def owait(obuf, sem):
        pltpu.make_async_copy(
            obuf, out_hbm.at[pl.ds(0, 64)], sem).wait()

    nit = 123

    fetch(wid, ba, s_a)

    def it_body(k, carry):
        c0 = wid + 64 * k
        c1 = c0 + 32

        @pl.when(c0 < 7812)
        def _():
            fwait(c0, ba, s_a)

            @pl.when(c1 < 7812)
            def _():
                fetch(c1, bb, s_b)

            @pl.when(k > 0)
            def _():
                owait(oa, s_oa)
            transpose(ba, oa)
            owrite(c0, oa, s_oa)

        @pl.when(c1 < 7812)
        def _():
            fwait(c1, bb, s_b)
            nxt = c0 + 64

            @pl.when(nxt < 7812)
            def _():
                fetch(nxt, ba, s_a)

            @pl.when(k > 0)
            def _():
                owait(ob, s_ob)
            transpose(bb, ob)
            owrite(c1, ob, s_ob)
        return carry

    lax.fori_loop(0, nit, it_body, 0)

    owait(oa, s_oa)
    owait(ob, s_ob)


_sc_densify = pl.kernel(
    _tr_body,
    out_type=jax.ShapeDtypeStruct((_VOCAB // 2, 2 * _D), jnp.float32),
    mesh=plsc.VectorSubcoreMesh(core_axis_name="c", subcore_axis_name="s",
                                num_cores=_NC, num_subcores=_NS),
    scratch_types=[
        pltpu.VMEM((_D, 128), jnp.float32),
        pltpu.VMEM((_D, 128), jnp.float32),
        pltpu.VMEM((_D, 128), jnp.float32),
        pltpu.VMEM((_D, 128), jnp.float32),
        pltpu.SemaphoreType.DMA,
        pltpu.SemaphoreType.DMA,
        pltpu.SemaphoreType.DMA,
        pltpu.SemaphoreType.DMA,
    ],
    compiler_params=pltpu.CompilerParams(needs_layout_passes=False),
)


def _mlp_body(h_ref, w1_ref, b1_ref, w2_ref, b2_ref, o_ref):
    h = h_ref[...] * (1.0 / _L)
    z = jnp.dot(h, w1_ref[...], preferred_element_type=jnp.float32)
    z = jnp.maximum(z + b1_ref[...], 0.0)
    o_ref[...] = jnp.dot(z, w2_ref[...],
                         preferred_element_type=jnp.float32) + b2_ref[...]


_mlp_call = pl.pallas_call(
    _mlp_body,
    out_shape=jax.ShapeDtypeStruct((_B, _H), jnp.float32),
)


def kernel(x, emb_table, W1, b1, W2, b2):
    t2 = _sc_densify(emb_table.T)
    t_tail = emb_table[7812 * 128:].reshape(32, 2 * _D)
    t2 = jax.lax.dynamic_update_slice(t2, t_tail, (7812 * 64, 0))
    t8 = t2.reshape(2 * _VOCAB, _HW)
    sums = _sc_gather_sum(x, t8)
    w2p = jnp.zeros((_H, _H), jnp.float32).at[:, :2].set(W2)
    b2p = jnp.zeros((1, _H), jnp.float32).at[0, :2].set(b2)
    out = _mlp_call(sums, W1, b1.reshape(1, _H), w2p, b2p)
    return out[:, :2]


# speedup vs baseline: 1.5827x; 1.0109x over previous
import jax
import jax.numpy as jnp
from jax import lax
from jax.experimental import pallas as pl
from jax.experimental.pallas import tpu as pltpu
from jax.experimental.pallas import tpu_sc as plsc

_VOCAB = 1000000
_D = 64
_H = 128
_B = 4096
_L = 200

_NC = 2
_NS = 16
_NW = _NC * _NS
_RPT = _B // _NW
_F = 2 * _L
_HW = _D // 2
_CHUNKS = ((0, 128), (128, 128), (256, 128), (384, 16))


def _sc_body(idx_hbm, tab_hbm, out_hbm, idx_v, pa, pb, ra, rb, out_v,
             s_a, s_b):
    c = lax.axis_index("c")
    s = lax.axis_index("s")
    wid = s * _NC + c
    rbase = wid * _RPT

    pltpu.sync_copy(idx_hbm.at[pl.ds(rbase, _RPT)], idx_v)

    lane = lax.broadcasted_iota(jnp.int32, (16,), 0)

    def precompute(row, pv):
        def pg(g, carry):
            iv = idx_v[row, pl.ds(g * 16, 16)]
            dbl = iv + iv
            pos = (g * 32) + lane + lane
            plsc.store_scatter(pv, [pos], dbl)
            plsc.store_scatter(pv, [pos + 1], dbl + 1)
            return carry
        lax.fori_loop(0, _L // 16, pg, 0)
        iv = idx_v[row, pl.ds(_L - 16, 16)]
        dbl = iv + iv
        pos = (2 * (_L - 16)) + lane + lane
        plsc.store_scatter(pv, [pos], dbl)
        plsc.store_scatter(pv, [pos + 1], dbl + 1)
        return None

    def gathers(pv, buf, sem):
        for off, width in _CHUNKS:
            pltpu.async_copy(
                tab_hbm.at[pv.at[pl.ds(off, width)]],
                buf.at[pl.ds(off, width)], sem)

    def waits(pv, buf, sem):
        for off, width in _CHUNKS:
            pltpu.make_async_copy(
                tab_hbm.at[pv.at[pl.ds(off, width)]],
                buf.at[pl.ds(off, width)], sem).wait()

    zero = jnp.zeros((16,), jnp.float32)
    acc_init = (zero,) * 8

    def reduce_row(buf, accs):
        def rb(i, a):
            r0 = 4 * i
            return (
                a[0] + buf[r0, pl.ds(0, 16)],
                a[1] + buf[r0, pl.ds(16, 16)],
                a[2] + buf[r0 + 1, pl.ds(0, 16)],
                a[3] + buf[r0 + 1, pl.ds(16, 16)],
                a[4] + buf[r0 + 2, pl.ds(0, 16)],
                a[5] + buf[r0 + 2, pl.ds(16, 16)],
                a[6] + buf[r0 + 3, pl.ds(0, 16)],
                a[7] + buf[r0 + 3, pl.ds(16, 16)],
            )
        return lax.fori_loop(0, _L // 2, rb, accs)

    def store_row(row, accs):
        out_v[row, pl.ds(0, 16)] = accs[0] + accs[4]
        out_v[row, pl.ds(16, 16)] = accs[1] + accs[5]
        out_v[row, pl.ds(32, 16)] = accs[2] + accs[6]
        out_v[row, pl.ds(48, 16)] = accs[3] + accs[7]

    precompute(0, pa)
    gathers(pa, ra, s_a)
    precompute(1, pb)
    gathers(pb, rb, s_b)

    nbb = _RPT // 2

    def iter_body(bb, carry):
        r0 = 2 * bb
        r1 = r0 + 1
        more = bb < nbb - 1

        waits(pa, ra, s_a)
        acc = reduce_row(ra, acc_init)

        @pl.when(more)
        def _():
            precompute(r0 + 2, pa)
            gathers(pa, ra, s_a)

        store_row(r0, acc)

        waits(pb, rb, s_b)
        acc = reduce_row(rb, acc_init)

        @pl.when(more)
        def _():
            precompute(r1 + 2, pb)
            gathers(pb, rb, s_b)

        store_row(r1, acc)
        return carry

    lax.fori_loop(0, nbb, iter_body, 0)

    pltpu.sync_copy(out_v, out_hbm.at[pl.ds(rbase, _RPT)])


_sc_gather_sum = pl.kernel(
    _sc_body,
    out_type=jax.ShapeDtypeStruct((_B, _D), jnp.float32),
    mesh=plsc.VectorSubcoreMesh(core_axis_name="c", subcore_axis_name="s",
                                num_cores=_NC, num_subcores=_NS),
    scratch_types=[
        pltpu.VMEM((_RPT, _L), jnp.int32),
        pltpu.VMEM((_F,), jnp.int32),
        pltpu.VMEM((_F,), jnp.int32),
        pltpu.VMEM((_F, _HW), jnp.float32),
        pltpu.VMEM((_F, _HW), jnp.float32),
        pltpu.VMEM((_RPT, _D), jnp.float32),
        pltpu.SemaphoreType.DMA,
        pltpu.SemaphoreType.DMA,
    ],
    compiler_params=pltpu.CompilerParams(use_tc_tiling_on_sc=False,
                                         needs_layout_passes=False),
)


_NBLK = 7813


def _tr_body(tabt_hbm, out_hbm, ba, bb, oa, ob, s_a, s_b, s_oa, s_ob):
    c = lax.axis_index("c")
    s = lax.axis_index("s")
    wid = s * _NC + c

    lane = lax.broadcasted_iota(jnp.int32, (16,), 0)

    def fetch(sb, buf, sem):
        pltpu.async_copy(tabt_hbm.at[:, pl.ds(sb * 256, 256)], buf, sem)

    def fwait(buf, sem):
        pltpu.make_async_copy(
            tabt_hbm.at[:, pl.ds(0, 256)], buf, sem).wait()

    def transpose(buf, obuf, co, ro):
        dims = [lane + 16 * m for m in range(4)]
        for sft in range(16):
            rot = (lane + sft) & 15
            st_r = lax.shift_right_logical(rot, 1)
            st_c = (rot & 1) * 64 + lane
            stc = [st_c + 16 * m for m in range(4)]

            @plsc.parallel_loop(0, 8, unroll=4)
            def _(e0g, rot=rot, st_r=st_r, stc=stc):
                ld_c = rot + e0g * 16 + co
                st_rr = st_r + e0g * 8 + ro
                for m in range(4):
                    v = plsc.load_gather(buf, [dims[m], ld_c])
                    plsc.store_scatter(obuf, [st_rr, stc[m]], v)

    def owrite(sb, obuf, sem):
        pltpu.async_copy(obuf, out_hbm.at[pl.ds(sb * 128, 128)], sem)

    def owait(obuf, sem):
        pltpu.make_async_copy(
            obuf, out_hbm.at[pl.ds(0, 128)], sem).wait()

    nsb = 3906
    nit = 62

    fetch(wid, ba, s_a)

    def it_body(k, carry):
        c0 = wid + 64 * k
        c1 = c0 + 32

        @pl.when(c0 < nsb)
        def _():
            fwait(ba, s_a)

            @pl.when(c1 < nsb)
            def _():
                fetch(c1, bb, s_b)

            @pl.when(k > 0)
            def _():
                owait(oa, s_oa)
            transpose(ba, oa, 0, 0)
            transpose(ba, oa, 128, 64)
            owrite(c0, oa, s_oa)

        @pl.when(c1 < nsb)
        def _():
            fwait(bb, s_b)
            nxt = c0 + 64

            @pl.when(nxt < nsb)
            def _():
                fetch(nxt, ba, s_a)

            @pl.when(k > 0)
            def _():
                owait(ob, s_ob)
            transpose(bb, ob, 0, 0)
            transpose(bb, ob, 128, 64)
            owrite(c1, ob, s_ob)
        return carry

    lax.fori_loop(0, nit, it_body, 0)

    owait(oa, s_oa)
    owait(ob, s_ob)


_sc_densify = pl.kernel(
    _tr_body,
    out_type=jax.ShapeDtypeStruct((_VOCAB // 2, 2 * _D), jnp.float32),
    mesh=plsc.VectorSubcoreMesh(core_axis_name="c", subcore_axis_name="s",
                                num_cores=_NC, num_subcores=_NS),
    scratch_types=[
        pltpu.VMEM((_D, 256), jnp.float32),
        pltpu.VMEM((_D, 256), jnp.float32),
        pltpu.VMEM((128, 128), jnp.float32),
        pltpu.VMEM((128, 128), jnp.float32),
        pltpu.SemaphoreType.DMA,
        pltpu.SemaphoreType.DMA,
        pltpu.SemaphoreType.DMA,
        pltpu.SemaphoreType.DMA,
    ],
    compiler_params=pltpu.CompilerParams(needs_layout_passes=False),
)


def _mlp_body(h_ref, w1_ref, b1_ref, w2_ref, b2_ref, o_ref):
    h = h_ref[...] * (1.0 / _L)
    z = jnp.dot(h, w1_ref[...], preferred_element_type=jnp.float32)
    z = jnp.maximum(z + b1_ref[...], 0.0)
    o_ref[...] = jnp.dot(z, w2_ref[...],
                         preferred_element_type=jnp.float32) + b2_ref[...]


_mlp_call = pl.pallas_call(
    _mlp_body,
    out_shape=jax.ShapeDtypeStruct((_B, _H), jnp.float32),
)


def kernel(x, emb_table, W1, b1, W2, b2):
    t2 = _sc_densify(emb_table.T)
    t_tail = emb_table[7812 * 128:].reshape(32, 2 * _D)
    t2 = jax.lax.dynamic_update_slice(t2, t_tail, (7812 * 64, 0))
    t8 = t2.reshape(2 * _VOCAB, _HW)
    sums = _sc_gather_sum(x, t8)
    w2p = jnp.zeros((_H, _H), jnp.float32).at[:, :2].set(W2)
    b2p = jnp.zeros((1, _H), jnp.float32).at[0, :2].set(b2)
    out = _mlp_call(sums, W1, b1.reshape(1, _H), w2p, b2p)
    return out[:, :2]
